# Initial kernel scaffold; baseline (speedup 1.0000x reference)
import jax
import jax.numpy as jnp
from jax.experimental import pallas as pl


def kernel(node_features, adj_mat, W1, a1_src, a1_dst, W2, a2_src, a2_dst, W3, a3_src, a3_dst, W4, a4_src, a4_dst):
    raise NotImplementedError("write your pallas kernel here")


# speedup vs baseline: 3.3156x; 3.3156x over previous
import functools

import jax
import jax.numpy as jnp
from jax.experimental import pallas as pl

_ALPHA = 0.2


def _pre_kernel(h, fh, x_ref, wf_ref, asrc_ref, adst_ref,
                s_ref, p_ref, q_ref, t_t_ref, u_t_ref, v_t_ref, whe_ref):
    n = x_ref.shape[0]
    wh = jnp.dot(x_ref[...], wf_ref[...], preferred_element_type=jnp.float32)
    s = jnp.dot(wh, asrc_ref[...], preferred_element_type=jnp.float32)
    t = jnp.dot(wh, adst_ref[...], preferred_element_type=jnp.float32)
    s_ref[...] = s
    p_ref[...] = jnp.exp(s)
    q_ref[...] = jnp.exp(_ALPHA * s)
    pad = t_t_ref.shape[0] - h
    t_t = jnp.concatenate([t.T, jnp.zeros((pad, n), jnp.float32)], axis=0)
    t_t_ref[...] = t_t
    u_t_ref[...] = jnp.exp(t_t)
    v_t_ref[...] = jnp.exp(_ALPHA * t_t)
    ones = jnp.ones((n, 1), jnp.float32)
    whe_ref[...] = jnp.concatenate(
        [jnp.concatenate([wh[:, i * fh:(i + 1) * fh], ones], axis=1)
         for i in range(h)], axis=1)


def _layer_kernel(h, fh, last,
                  adj_ref, s_ref, p_ref, q_ref, t_t_ref, u_t_ref, v_t_ref,
                  whe_ref, out_ref):
    m = adj_ref[...]
    outs = []
    for i in range(h):
        s_c = s_ref[:, i:i + 1]
        e = s_c + t_t_ref[i:i + 1, :]
        w = jnp.where(e > 0,
                      p_ref[:, i:i + 1] * u_t_ref[i:i + 1, :],
                      q_ref[:, i:i + 1] * v_t_ref[i:i + 1, :])
        w = w * m
        nd = jnp.dot(w, whe_ref[:, i * (fh + 1):(i + 1) * (fh + 1)],
                     preferred_element_type=jnp.float32)
        outs.append(nd[:, :fh] / nd[:, fh:fh + 1])
    if not last:
        o = jnp.concatenate(outs, axis=1)
        out_ref[...] = jnp.where(o > 0, o, jnp.exp(o) - 1.0)
    else:
        o = outs[0]
        for x in outs[1:]:
            o = o + x
        o = jnp.maximum(o * (1.0 / h), 0.0)
        part = jnp.sum(o, axis=0, keepdims=True)

        @pl.when(pl.program_id(0) == 0)
        def _():
            out_ref[...] = jnp.zeros_like(out_ref)

        out_ref[...] += part


def _gat_layer(x, adj, wf, asrc_bd, adst_bd, h, fh, last, block_i=256):
    n = x.shape[0]
    hp = 8
    pre = pl.pallas_call(
        functools.partial(_pre_kernel, h, fh),
        out_shape=(
            jax.ShapeDtypeStruct((n, h), jnp.float32),
            jax.ShapeDtypeStruct((n, h), jnp.float32),
            jax.ShapeDtypeStruct((n, h), jnp.float32),
            jax.ShapeDtypeStruct((hp, n), jnp.float32),
            jax.ShapeDtypeStruct((hp, n), jnp.float32),
            jax.ShapeDtypeStruct((hp, n), jnp.float32),
            jax.ShapeDtypeStruct((n, h * (fh + 1)), jnp.float32),
        ),
    )
    s, p, q, t_t, u_t, v_t, whe = pre(x, wf, asrc_bd, adst_bd)

    nb = n // block_i
    if last:
        out_sds = jax.ShapeDtypeStruct((1, fh), jnp.float32)
        out_spec = pl.BlockSpec((1, fh), lambda i: (0, 0))
    else:
        out_sds = jax.ShapeDtypeStruct((n, h * fh), jnp.float32)
        out_spec = pl.BlockSpec((block_i, h * fh), lambda i: (i, 0))
    full = lambda a: pl.BlockSpec(a.shape, lambda i: (0,) * a.ndim)
    rows = lambda a: pl.BlockSpec((block_i, a.shape[1]), lambda i: (i, 0))
    return pl.pallas_call(
        functools.partial(_layer_kernel, h, fh, last),
        grid=(nb,),
        in_specs=[rows(adj), rows(s), rows(p), rows(q),
                  full(t_t), full(u_t), full(v_t), full(whe)],
        out_specs=out_spec,
        out_shape=out_sds,
    )(adj, s, p, q, t_t, u_t, v_t, whe)


def _block_diag(a):
    h, fh = a.shape
    eye = jnp.eye(h, dtype=a.dtype)
    return (a[:, :, None] * eye[:, None, :]).reshape(h * fh, h)


def kernel(node_features, adj_mat,
           W1, a1_src, a1_dst,
           W2, a2_src, a2_dst,
           W3, a3_src, a3_dst,
           W4, a4_src, a4_dst):
    x = node_features
    layers = [
        (W1, a1_src, a1_dst, False),
        (W2, a2_src, a2_dst, False),
        (W3, a3_src, a3_dst, False),
        (W4, a4_src, a4_dst, True),
    ]
    for w, a_s, a_d, last in layers:
        fin, h, fh = w.shape
        x = _gat_layer(x, adj_mat, w.reshape(fin, h * fh),
                       _block_diag(a_s), _block_diag(a_d), h, fh, last)
    return x.reshape(-1)


# speedup vs baseline: 3.9362x; 1.1872x over previous
import functools

import jax
import jax.numpy as jnp
from jax.experimental import pallas as pl

_ALPHA = 0.2


def _pre_kernel(h, fh, x_ref, wf_ref, asrc_ref, adst_ref,
                s_ref, p_ref, q_ref, t_t_ref, u_t_ref, v_t_ref, whe_ref):
    n = x_ref.shape[0]
    wh = jnp.dot(x_ref[...], wf_ref[...], preferred_element_type=jnp.float32)
    s = jnp.dot(wh, asrc_ref[...], preferred_element_type=jnp.float32)
    t = jnp.dot(wh, adst_ref[...], preferred_element_type=jnp.float32)
    cast = lambda x: x.astype(s_ref.dtype)
    s_ref[...] = cast(s)
    p_ref[...] = cast(jnp.exp(s))
    q_ref[...] = cast(jnp.exp(_ALPHA * s))
    pad = t_t_ref.shape[0] - h
    t_t = jnp.concatenate([t.T, jnp.zeros((pad, n), jnp.float32)], axis=0)
    t_t_ref[...] = cast(t_t)
    u_t_ref[...] = cast(jnp.exp(t_t))
    v_t_ref[...] = cast(jnp.exp(_ALPHA * t_t))
    ones = jnp.ones((n, 1), jnp.float32)
    whe_ref[...] = cast(jnp.concatenate(
        [jnp.concatenate([wh[:, i * fh:(i + 1) * fh], ones], axis=1)
         for i in range(h)], axis=1))


def _layer_kernel(h, fh, last,
                  adj_ref, s_ref, p_ref, q_ref, t_t_ref, u_t_ref, v_t_ref,
                  whe_ref, out_ref):
    m = adj_ref[...].astype(s_ref.dtype)
    outs = []
    for i in range(h):
        s_c = s_ref[:, i:i + 1]
        e = s_c + t_t_ref[i:i + 1, :]
        w = jnp.where(e > 0,
                      p_ref[:, i:i + 1] * u_t_ref[i:i + 1, :],
                      q_ref[:, i:i + 1] * v_t_ref[i:i + 1, :])
        w = w * m
        nd = jnp.dot(w, whe_ref[:, i * (fh + 1):(i + 1) * (fh + 1)],
                     preferred_element_type=jnp.float32)
        outs.append(nd[:, :fh] / nd[:, fh:fh + 1])
    if not last:
        o = jnp.concatenate(outs, axis=1)
        out_ref[...] = jnp.where(o > 0, o, jnp.exp(o) - 1.0)
    else:
        o = outs[0]
        for x in outs[1:]:
            o = o + x
        o = jnp.maximum(o * (1.0 / h), 0.0)
        part = jnp.sum(o, axis=0, keepdims=True)

        @pl.when(pl.program_id(0) == 0)
        def _():
            out_ref[...] = jnp.zeros_like(out_ref)

        out_ref[...] += part


def _gat_layer(x, adj, wf, asrc_bd, adst_bd, h, fh, last, block_i=256):
    n = x.shape[0]
    hp = 8
    wdt = jnp.bfloat16
    pre = pl.pallas_call(
        functools.partial(_pre_kernel, h, fh),
        out_shape=(
            jax.ShapeDtypeStruct((n, h), wdt),
            jax.ShapeDtypeStruct((n, h), wdt),
            jax.ShapeDtypeStruct((n, h), wdt),
            jax.ShapeDtypeStruct((hp, n), wdt),
            jax.ShapeDtypeStruct((hp, n), wdt),
            jax.ShapeDtypeStruct((hp, n), wdt),
            jax.ShapeDtypeStruct((n, h * (fh + 1)), wdt),
        ),
    )
    s, p, q, t_t, u_t, v_t, whe = pre(x, wf, asrc_bd, adst_bd)

    nb = n // block_i
    if last:
        out_sds = jax.ShapeDtypeStruct((1, fh), jnp.float32)
        out_spec = pl.BlockSpec((1, fh), lambda i: (0, 0))
    else:
        out_sds = jax.ShapeDtypeStruct((n, h * fh), jnp.float32)
        out_spec = pl.BlockSpec((block_i, h * fh), lambda i: (i, 0))
    full = lambda a: pl.BlockSpec(a.shape, lambda i: (0,) * a.ndim)
    rows = lambda a: pl.BlockSpec((block_i, a.shape[1]), lambda i: (i, 0))
    return pl.pallas_call(
        functools.partial(_layer_kernel, h, fh, last),
        grid=(nb,),
        in_specs=[rows(adj), rows(s), rows(p), rows(q),
                  full(t_t), full(u_t), full(v_t), full(whe)],
        out_specs=out_spec,
        out_shape=out_sds,
    )(adj, s, p, q, t_t, u_t, v_t, whe)


def _block_diag(a):
    h, fh = a.shape
    eye = jnp.eye(h, dtype=a.dtype)
    return (a[:, :, None] * eye[:, None, :]).reshape(h * fh, h)


def kernel(node_features, adj_mat,
           W1, a1_src, a1_dst,
           W2, a2_src, a2_dst,
           W3, a3_src, a3_dst,
           W4, a4_src, a4_dst):
    x = node_features
    layers = [
        (W1, a1_src, a1_dst, False),
        (W2, a2_src, a2_dst, False),
        (W3, a3_src, a3_dst, False),
        (W4, a4_src, a4_dst, True),
    ]
    for w, a_s, a_d, last in layers:
        fin, h, fh = w.shape
        x = _gat_layer(x, adj_mat, w.reshape(fin, h * fh),
                       _block_diag(a_s), _block_diag(a_d), h, fh, last)
    return x.reshape(-1)


# speedup vs baseline: 4.3084x; 1.0946x over previous
import functools

import jax
import jax.numpy as jnp
from jax.experimental import pallas as pl
from jax.experimental.pallas import tpu as pltpu

_ALPHA = 0.2
_WDT = jnp.bfloat16


def _precompute(lc, layers, x, wf_ref, ws_ref, wt_ref,
                ns_ref, p_ref, q_ref, t_t_ref, u_t_ref, v_t_ref, whe_ref):
    h, fh, fin = layers[lc]
    n = x.shape[0]
    wh = jnp.dot(x, wf_ref[...], preferred_element_type=jnp.float32)
    s = jnp.dot(x, ws_ref[...], preferred_element_type=jnp.float32)
    t = jnp.dot(x, wt_ref[...], preferred_element_type=jnp.float32)
    ns_ref[:, :h] = (-s).astype(_WDT)
    p_ref[:, :h] = jnp.exp(s).astype(_WDT)
    q_ref[:, :h] = jnp.exp(_ALPHA * s).astype(_WDT)
    t_t = t.T
    t_t_ref[:h, :] = t_t.astype(_WDT)
    u_t_ref[:h, :] = jnp.exp(t_t).astype(_WDT)
    v_t_ref[:h, :] = jnp.exp(_ALPHA * t_t).astype(_WDT)
    ones = jnp.ones((n, 1), jnp.float32)
    whe = jnp.concatenate(
        [jnp.concatenate([wh[:, i * fh:(i + 1) * fh], ones], axis=1)
         for i in range(h)], axis=1)
    whe_ref[:, :h * (fh + 1)] = whe.astype(_WDT)


def _attend(lc, layers, bi, i, m, ns_ref, p_ref, q_ref,
            t_t_ref, u_t_ref, v_t_ref, whe_ref):
    h, fh, _ = layers[lc]
    sl = pl.ds(i * bi, bi)
    outs = []
    for k in range(h):
        ns_c = ns_ref[sl, k:k + 1]
        pos = t_t_ref[k:k + 1, :] > ns_c
        w = jnp.where(pos,
                      p_ref[sl, k:k + 1] * u_t_ref[k:k + 1, :],
                      q_ref[sl, k:k + 1] * v_t_ref[k:k + 1, :])
        w = w * m
        nd = jnp.dot(w, whe_ref[:, k * (fh + 1):(k + 1) * (fh + 1)],
                     preferred_element_type=jnp.float32)
        outs.append(nd[:, :fh] / nd[:, fh:fh + 1])
    return outs


def _mega_kernel(layers, bi,
                 adj_ref, x0_ref, wf_refs, ws_refs, wt_refs, out_ref,
                 mask_ref, x_ref, ns_ref, p_ref, q_ref,
                 t_t_ref, u_t_ref, v_t_ref, whe_ref):
    l = pl.program_id(0)
    i = pl.program_id(1)

    for lc in range(len(layers)):
        h, fh, fin = layers[lc]
        last = lc == len(layers) - 1

        @pl.when(jnp.logical_and(l == lc, i == 0))
        def _(lc=lc, fin=fin):
            x = x0_ref[...] if lc == 0 else x_ref[:, :fin]
            _precompute(lc, layers, x, wf_refs[lc], ws_refs[lc], wt_refs[lc],
                        ns_ref, p_ref, q_ref, t_t_ref, u_t_ref, v_t_ref,
                        whe_ref)

        @pl.when(l == lc)
        def _(lc=lc, h=h, fh=fh, last=last):
            sl = pl.ds(i * bi, bi)
            if lc == 0:
                m = adj_ref[...].astype(_WDT)
                mask_ref[sl, :] = m
            else:
                m = mask_ref[sl, :]
            outs = _attend(lc, layers, bi, i, m, ns_ref, p_ref, q_ref,
                           t_t_ref, u_t_ref, v_t_ref, whe_ref)
            if not last:
                o = jnp.concatenate(outs, axis=1)
                x_ref[sl, :h * fh] = jnp.where(o > 0, o, jnp.exp(o) - 1.0)
            else:
                o = outs[0]
                for x in outs[1:]:
                    o = o + x
                o = jnp.maximum(o * (1.0 / h), 0.0)
                part = jnp.sum(o, axis=0, keepdims=True)

                @pl.when(i == 0)
                def _():
                    out_ref[...] = jnp.zeros_like(out_ref)

                out_ref[...] += part


def kernel(node_features, adj_mat,
           W1, a1_src, a1_dst,
           W2, a2_src, a2_dst,
           W3, a3_src, a3_dst,
           W4, a4_src, a4_dst):
    n = node_features.shape[0]
    bi = min(256, n)
    nb = n // bi
    params = ((W1, a1_src, a1_dst), (W2, a2_src, a2_dst),
              (W3, a3_src, a3_dst), (W4, a4_src, a4_dst))
    layers = tuple((w.shape[1], w.shape[2], w.shape[0]) for w, _, _ in params)
    wfs, wss, wts = [], [], []
    for w, a_s, a_d in params:
        fin, h, fh = w.shape
        wf = w.reshape(fin, h * fh)
        wfs.append(wf)
        eye = jnp.eye(h, dtype=w.dtype)
        bd_s = (a_s[:, :, None] * eye[:, None, :]).reshape(h * fh, h)
        bd_d = (a_d[:, :, None] * eye[:, None, :]).reshape(h * fh, h)
        wss.append(wf @ bd_s)
        wts.append(wf @ bd_d)

    full = lambda a: pl.BlockSpec(a.shape, lambda l, i: (0,) * a.ndim)
    max_whe = max(h * (fh + 1) for h, fh, _ in layers)
    fh_last = layers[-1][1]
    out = pl.pallas_call(
        functools.partial(_mega_kernel, layers, bi),
        grid=(len(layers), nb),
        in_specs=[
            pl.BlockSpec((bi, n), lambda l, i: (jnp.where(l == 0, i, 0), 0)),
            full(node_features),
            [full(w) for w in wfs],
            [full(w) for w in wss],
            [full(w) for w in wts],
        ],
        out_specs=pl.BlockSpec((1, fh_last), lambda l, i: (0, 0)),
        out_shape=jax.ShapeDtypeStruct((1, fh_last), jnp.float32),
        scratch_shapes=[
            pltpu.VMEM((n, n), _WDT),
            pltpu.VMEM((n, 16), jnp.float32),
            pltpu.VMEM((n, 8), _WDT),
            pltpu.VMEM((n, 8), _WDT),
            pltpu.VMEM((n, 8), _WDT),
            pltpu.VMEM((8, n), _WDT),
            pltpu.VMEM((8, n), _WDT),
            pltpu.VMEM((8, n), _WDT),
            pltpu.VMEM((n, max_whe), _WDT),
        ],
    )(adj_mat, node_features, wfs, wss, wts)
    return out.reshape(-1)
